# baseline (speedup 1.0000x reference)
import functools

import numpy as _np

import jax
import jax.numpy as jnp
from jax import lax
from jax.experimental import pallas as pl
from jax.experimental.pallas import tpu as pltpu
from jax.experimental.pallas import tpu_sc as plsc

_NC = 2
_NS = 16
_NW = _NC * _NS
_L = 16
_CA = 80
_CB = 96

_F32 = jnp.float32
_BF16 = jnp.bfloat16
_I32 = jnp.int32
_ILV = plsc.PackFormat.INTERLEAVED


def _mm_bias_body(a_ref, w_ref, b_ref, o_ref):
    o_ref[...] = (
        jnp.dot(a_ref[...], w_ref[...], preferred_element_type=_F32)
        + b_ref[...]
    ).astype(o_ref.dtype)


def _rows_mm_bias(a, w, b2d, block_rows, out_dtype):
    m, k = a.shape
    h = w.shape[1]
    assert m % block_rows == 0
    return pl.pallas_call(
        _mm_bias_body,
        grid=(m // block_rows,),
        in_specs=[
            pl.BlockSpec((block_rows, k), lambda i: (i, 0)),
            pl.BlockSpec((k, h), lambda i: (0, 0)),
            pl.BlockSpec((1, h), lambda i: (0, 0)),
        ],
        out_specs=pl.BlockSpec((block_rows, h), lambda i: (i, 0)),
        out_shape=jax.ShapeDtypeStruct((m, h), out_dtype),
    )(a, w, b2d)


def _mm_pack_body(a_ref, w_ref, b_ref, o_ref):
    res = (jnp.dot(a_ref[...], w_ref[...], preferred_element_type=_F32)
           + b_ref[...])
    res3 = res.reshape(res.shape[0] // 2, 2, res.shape[1])
    lo = lax.bitcast_convert_type(res3[:, 0, :].astype(_BF16), jnp.uint16)
    hi = lax.bitcast_convert_type(res3[:, 1, :].astype(_BF16), jnp.uint16)
    word = lo.astype(jnp.uint32) | (hi.astype(jnp.uint32) << 16)
    o_ref[...] = lax.bitcast_convert_type(word, _I32)


def _rows_mm_bias_pack(a, w, b2d, block_rows):
    m, k = a.shape
    h = w.shape[1]
    assert m % block_rows == 0 and block_rows % 2 == 0
    return pl.pallas_call(
        _mm_pack_body,
        grid=(m // block_rows,),
        in_specs=[
            pl.BlockSpec((block_rows, k), lambda i: (i, 0)),
            pl.BlockSpec((k, h), lambda i: (0, 0)),
            pl.BlockSpec((1, h), lambda i: (0, 0)),
        ],
        out_specs=pl.BlockSpec((block_rows // 2, h), lambda i: (i, 0)),
        out_shape=jax.ShapeDtypeStruct((m // 2, h), _I32),
    )(a, w, b2d)


def _emb_body(x_ref, a0_ref, a1_ref, w1_ref, w2_ref, b1_ref,
              wps_ref, wpo_ref, g1_ref, g2_ref):
    aggr = a0_ref[0] + a1_ref[0]
    emb = jnp.maximum(
        jnp.dot(x_ref[...], w1_ref[...], preferred_element_type=_F32)
        + jnp.dot(aggr, w2_ref[...], preferred_element_type=_F32)
        + b1_ref[...],
        0.0,
    )
    g1_ref[...] = jnp.dot(emb, wps_ref[...], preferred_element_type=_F32)
    g2_ref[...] = jnp.dot(emb, wpo_ref[...], preferred_element_type=_F32)


def _emb_g1_g2(x, aggr2, w1, w2, b1_2d, wps, wpo, block_rows):
    n, d = x.shape
    h = wps.shape[1]
    assert n % block_rows == 0
    full = lambda i: (0, 0)
    row_blk = lambda i: (i, 0)
    return pl.pallas_call(
        _emb_body,
        grid=(n // block_rows,),
        in_specs=[
            pl.BlockSpec((block_rows, d), row_blk),
            pl.BlockSpec((1, block_rows, d), lambda i: (0, i, 0)),
            pl.BlockSpec((1, block_rows, d), lambda i: (1, i, 0)),
            pl.BlockSpec((d, d), full),
            pl.BlockSpec((d, d), full),
            pl.BlockSpec((1, d), full),
            pl.BlockSpec((d, h), full),
            pl.BlockSpec((d, h), full),
        ],
        out_specs=[
            pl.BlockSpec((block_rows, h), row_blk),
            pl.BlockSpec((block_rows, h), row_blk),
        ],
        out_shape=[
            jax.ShapeDtypeStruct((n, h), _F32),
            jax.ShapeDtypeStruct((n, h), _F32),
        ],
    )(x, aggr2, aggr2, w1, w2, b1_2d, wps, wpo)


def _make_stage_a(n_nodes, e, d):
    assert e % _CA == 0
    nch = e // _CA
    nka = (nch + _NW - 1) // _NW
    npos = nka + 2
    npos += (-npos) % 3
    kcol = d // _L
    nfull = n_nodes // _CA
    ntail = n_nodes % _CA

    @functools.partial(
        pl.kernel,
        out_type=jax.ShapeDtypeStruct((2 * n_nodes, d), _F32),
        mesh=plsc.VectorSubcoreMesh(core_axis_name="c", subcore_axis_name="s"),
        scratch_types=[
            pltpu.VMEM((3, _CA // 2, d), _I32),
            pltpu.VMEM((3, _CA, d), _F32),
            pltpu.VMEM((3, _CA), _I32),
            pltpu.VMEM((3, _CA), _I32),
            pltpu.VMEM_SHARED((n_nodes, d), _F32),
        ] + [pltpu.SemaphoreType.DMA] * 12,
        compiler_params=pltpu.CompilerParams(needs_layout_passes=False),
    )
    def stage_a(x_hbm, src_hbm, dst_hbm, p_hbm, z_hbm, out_hbm,
                pbuf, xbuf, sidx, didx, aggr_s, *sems):
        semi = sems[0:3]
        semp = sems[3:6]
        semg = sems[6:9]
        semsc = sems[9:12]
        cid = lax.axis_index("c")
        sid = lax.axis_index("s")
        wid = sid * _NC + cid

        pltpu.sync_copy(z_hbm, xbuf.at[0])

        def zbody(kk, carry):
            chunk = sid + kk * _NS

            @pl.when(chunk < nfull)
            def _():
                pltpu.sync_copy(xbuf.at[0],
                                aggr_s.at[pl.ds(chunk * _CA, _CA), :])

            return carry

        lax.fori_loop(0, (nfull + _NS - 1) // _NS, zbody, 0)
        if ntail:
            @pl.when(sid == _NS - 1)
            def _():
                pltpu.sync_copy(xbuf.at[0, pl.ds(0, ntail), :],
                                aggr_s.at[pl.ds(nfull * _CA, ntail), :])
        plsc.subcore_barrier()

        def d_sidx(kk, u):
            base = (kk * _NW + wid) * _CA
            return pltpu.make_async_copy(
                src_hbm.at[pl.ds(base, _CA)], sidx.at[u], semi[u])

        def d_didx(kk, u):
            base = (kk * _NW + wid) * _CA
            return pltpu.make_async_copy(
                dst_hbm.at[pl.ds(base, _CA)], didx.at[u], semi[u])

        def d_p(kk, u):
            base = (kk * _NW + wid) * (_CA // 2)
            return pltpu.make_async_copy(
                p_hbm.at[pl.ds(base, _CA // 2), :], pbuf.at[u], semp[u])

        def d_gather(u):
            return pltpu.make_async_copy(
                x_hbm.at[sidx.at[u]], xbuf.at[u], semg[u])

        def d_scat(u):
            return pltpu.make_async_copy(
                xbuf.at[u], aggr_s.at[didx.at[u]], semsc[u])

        def s1(kk, u):
            c = kk * _NW + wid
            valid = c < nch

            @pl.when((kk >= 3) & valid)
            def _():
                d_scat(u).wait()

            @pl.when(valid)
            def _():
                d_sidx(kk, u).start()
                d_didx(kk, u).start()
                d_p(kk, u).start()

        def g(kk, u):
            c = kk * _NW + wid

            @pl.when((kk >= 0) & (c < nch))
            def _():
                d_sidx(kk, u).wait()
                d_didx(kk, u).wait()
                d_gather(u).start()

        def f(kk, u):
            c = kk * _NW + wid

            @pl.when((kk >= 0) & (c < nch))
            def _():
                d_gather(u).wait()
                d_p(kk, u).wait()

                def crow(rp, carry2):
                    for k in range(kcol):
                        sl = pl.ds(k * _L, _L)
                        pa, pb = plsc.unpack(
                            plsc.bitcast(pbuf[u, rp, sl], _BF16), format=_ILV)
                        xbuf[u, 2 * rp, sl] = jnp.maximum(
                            xbuf[u, 2 * rp, sl] + pa, 0.0)
                        xbuf[u, 2 * rp + 1, sl] = jnp.maximum(
                            xbuf[u, 2 * rp + 1, sl] + pb, 0.0)
                    return carry2

                lax.fori_loop(0, _CA // 2, crow, 0)
                pltpu.async_copy(xbuf.at[u], aggr_s.at[didx.at[u]],
                                 semsc[u], add=True)

        def mainloop(q, carry):
            i0 = q * 3
            for uu in range(3):
                i = i0 + uu
                g(i - 1, (uu + 2) % 3)
                s1(i, uu)
                f(i - 2, (uu + 1) % 3)
            return carry

        lax.fori_loop(0, npos // 3, mainloop, 0)
        for u in range(3):
            d_scat(u).wait()
        plsc.subcore_barrier()

        def obody(kk, carry):
            chunk = sid + kk * _NS

            @pl.when(chunk < nfull)
            def _():
                pltpu.sync_copy(
                    aggr_s.at[pl.ds(chunk * _CA, _CA), :],
                    out_hbm.at[pl.ds(cid * n_nodes + chunk * _CA, _CA), :])

            return carry

        lax.fori_loop(0, (nfull + _NS - 1) // _NS, obody, 0)
        if ntail:
            @pl.when(sid == _NS - 1)
            def _():
                pltpu.sync_copy(
                    aggr_s.at[pl.ds(nfull * _CA, ntail), :],
                    out_hbm.at[pl.ds(cid * n_nodes + nfull * _CA, ntail), :])

    return stage_a


def _make_stage_b(t, d):
    nfullb = t // _CB
    tb = t % _CB
    assert tb % 8 == 0
    ntotb = nfullb + (1 if tb else 0)
    nkb = (ntotb + _NW - 1) // _NW
    npos = nkb + 2
    npos += (-npos) % 3
    kcol = d // _L
    tailw = nfullb % _NW
    tail_slot = ((ntotb - 1 - tailw) // _NW) % 3

    @functools.partial(
        pl.kernel,
        out_type=jax.ShapeDtypeStruct((t,), _F32),
        mesh=plsc.VectorSubcoreMesh(core_axis_name="c", subcore_axis_name="s"),
        scratch_types=[
            pltpu.VMEM((3, _CB, d), _F32),
            pltpu.VMEM((3, _CB, d), _F32),
            pltpu.VMEM((3, _CB, d), _F32),
            pltpu.VMEM((3, _CB), _I32),
            pltpu.VMEM((3, _CB), _I32),
            pltpu.VMEM((3, _CB), _F32),
            pltpu.VMEM((d + _L,), _F32),
        ] + [pltpu.SemaphoreType.DMA] * 15,
        compiler_params=pltpu.CompilerParams(needs_layout_passes=False),
    )
    def stage_b(g1_hbm, g2_hbm, s_hbm, o_hbm, r_hbm, wb_hbm, out_hbm,
                g1buf, g2buf, rbuf, sidx, oidx, obuf, wbuf, *sems):
        semi = sems[0:3]
        semr = sems[3:6]
        semg1 = sems[6:9]
        semg2 = sems[9:12]
        semo = sems[12:15]
        cid = lax.axis_index("c")
        sid = lax.axis_index("s")
        wid = sid * _NC + cid

        pltpu.sync_copy(wb_hbm, wbuf)
        w2 = [wbuf[pl.ds(k * _L, _L)] for k in range(kcol)]
        b2v = wbuf[pl.ds(d, _L)]
        iota = lax.iota(_I32, _L)
        zero = jnp.zeros((_L,), _F32)

        def d_sidx(kk, u, n):
            base = (kk * _NW + wid) * _CB
            return pltpu.make_async_copy(
                s_hbm.at[pl.ds(base, n)],
                sidx.at[u, pl.ds(0, n)], semi[u])

        def d_oidx(kk, u, n):
            base = (kk * _NW + wid) * _CB
            return pltpu.make_async_copy(
                o_hbm.at[pl.ds(base, n)],
                oidx.at[u, pl.ds(0, n)], semi[u])

        def d_r(kk, u, n):
            base = (kk * _NW + wid) * _CB
            return pltpu.make_async_copy(
                r_hbm.at[pl.ds(base, n), :],
                rbuf.at[u, pl.ds(0, n), :], semr[u])

        def d_g1(u, n):
            return pltpu.make_async_copy(
                g1_hbm.at[sidx.at[u, pl.ds(0, n)]],
                g1buf.at[u, pl.ds(0, n), :], semg1[u])

        def d_g2(u, n):
            return pltpu.make_async_copy(
                g2_hbm.at[oidx.at[u, pl.ds(0, n)]],
                g2buf.at[u, pl.ds(0, n), :], semg2[u])

        def d_o(kk, u, n):
            base = (kk * _NW + wid) * _CB
            return pltpu.make_async_copy(
                obuf.at[u, pl.ds(0, n)],
                out_hbm.at[pl.ds(base, n)], semo[u])

        def both(cond_full, cond_tail, fn):
            if tb:
                @pl.when(cond_full)
                def _():
                    fn(_CB)

                @pl.when(cond_tail)
                def _():
                    fn(tb)
            else:
                @pl.when(cond_full)
                def _():
                    fn(_CB)

        def s1(kk, u):
            c = kk * _NW + wid

            @pl.when((kk >= 3) & (c < ntotb))
            def _():
                d_o(kk - 3, u, _CB).wait()

            def start(n):
                d_sidx(kk, u, n).start()
                d_oidx(kk, u, n).start()
                d_r(kk, u, n).start()

            both(c < nfullb, c == nfullb, start)

        def g(kk, u):
            ok = kk >= 0
            c = kk * _NW + wid

            def fire(n):
                d_sidx(kk, u, n).wait()
                d_oidx(kk, u, n).wait()
                d_g1(u, n).start()
                d_g2(u, n).start()

            both(ok & (c < nfullb), ok & (c == nfullb), fire)

        def f(kk, u):
            ok = kk >= 0
            c = kk * _NW + wid

            def waits(n):
                d_g1(u, n).wait()
                d_g2(u, n).wait()
                d_r(kk, u, n).wait()

            both(ok & (c < nfullb), ok & (c == nfullb), waits)

            @pl.when(ok & (c < ntotb))
            def _():
                nrows = jnp.where(c == nfullb, tb, _CB) if tb else _CB

                def ebody(row, s):
                    acc = zero
                    for k in range(kcol):
                        sl = pl.ds(k * _L, _L)
                        v = (g1buf[u, row, sl] + g2buf[u, row, sl]
                             + rbuf[u, row, sl])
                        acc = acc + jnp.maximum(v, 0.0) * w2[k]
                    tot = jnp.sum(acc)
                    lane = lax.rem(row, _L)
                    s = jnp.where(iota == lane, tot, s)

                    @pl.when(lane == _L - 1)
                    def _():
                        obuf[u, pl.ds(row - (_L - 1), _L)] = s

                    return jnp.where(lane == _L - 1, b2v, s)

                lax.fori_loop(0, nrows, ebody, b2v)

            both(ok & (c < nfullb), ok & (c == nfullb),
                 lambda n: d_o(kk, u, n).start())

        def mainloop(q, carry):
            i0 = q * 3
            for uu in range(3):
                i = i0 + uu
                g(i - 1, (uu + 2) % 3)
                s1(i, uu)
                f(i - 2, (uu + 1) % 3)
            return carry

        lax.fori_loop(0, npos // 3, mainloop, 0)
        for u in range(3):
            if tb and u == tail_slot:
                @pl.when(wid == tailw)
                def _():
                    d_o(0, u, tb).wait()

                @pl.when(wid != tailw)
                def _():
                    d_o(0, u, _CB).wait()
            else:
                d_o(0, u, _CB).wait()

    return stage_b


def kernel(x, edge_index, edge_attr, target_edges, target_edge_attrs,
           We, W1, W2, b1, Wp1, bp1, Wp2, bp2):
    n_nodes, d = x.shape
    h = Wp1.shape[1]
    e = edge_index.shape[1]
    t = target_edges.shape[1]

    ei = edge_index.astype(_I32)
    te = target_edges.astype(_I32)

    wp1_s = Wp1[:d]
    wp1_m = Wp1[d:d + target_edge_attrs.shape[1]]
    wp1_o = Wp1[d + target_edge_attrs.shape[1]:]

    p = _rows_mm_bias_pack(edge_attr, We, jnp.zeros((1, d), _F32),
                           block_rows=2000)
    r = _rows_mm_bias(target_edge_attrs, wp1_m, bp1.reshape(1, h),
                      block_rows=2000, out_dtype=_F32)

    stage_a = _make_stage_a(n_nodes, e, d)
    zeros_chunk = jnp.zeros((_CA, d), _F32)
    aggr2 = stage_a(x, ei[0], ei[1], p, zeros_chunk).reshape(2, n_nodes, d)

    g1, g2 = _emb_g1_g2(x, aggr2, W1, W2, b1.reshape(1, d),
                        wp1_s, wp1_o, block_rows=400)

    wb = jnp.concatenate([Wp2[:, 0], jnp.broadcast_to(bp2, (_L,))])
    stage_b = _make_stage_b(t, h)
    return stage_b(g1, g2, te[0], te[1], r, wb)


# speedup vs baseline: 1.2833x; 1.2833x over previous
import functools

import numpy as _np

import jax
import jax.numpy as jnp
from jax import lax
from jax.experimental import pallas as pl
from jax.experimental.pallas import tpu as pltpu
from jax.experimental.pallas import tpu_sc as plsc

_NC = 2
_NS = 16
_NW = _NC * _NS
_L = 16
_CA = 64
_CB = 96

_F32 = jnp.float32
_BF16 = jnp.bfloat16
_I32 = jnp.int32
_ILV = plsc.PackFormat.INTERLEAVED


def _mm_bias_body(a_ref, w_ref, b_ref, o_ref):
    o_ref[...] = (
        jnp.dot(a_ref[...], w_ref[...], preferred_element_type=_F32)
        + b_ref[...]
    ).astype(o_ref.dtype)


def _rows_mm_bias(a, w, b2d, block_rows, out_dtype):
    m, k = a.shape
    h = w.shape[1]
    assert m % block_rows == 0
    return pl.pallas_call(
        _mm_bias_body,
        grid=(m // block_rows,),
        in_specs=[
            pl.BlockSpec((block_rows, k), lambda i: (i, 0)),
            pl.BlockSpec((k, h), lambda i: (0, 0)),
            pl.BlockSpec((1, h), lambda i: (0, 0)),
        ],
        out_specs=pl.BlockSpec((block_rows, h), lambda i: (i, 0)),
        out_shape=jax.ShapeDtypeStruct((m, h), out_dtype),
    )(a, w, b2d)


def _mm_pack_body(a_ref, w_ref, b_ref, o_ref):
    res = (jnp.dot(a_ref[...], w_ref[...], preferred_element_type=_F32)
           + b_ref[...])
    res3 = res.reshape(res.shape[0] // 2, 2, res.shape[1])
    lo = lax.bitcast_convert_type(res3[:, 0, :].astype(_BF16), jnp.uint16)
    hi = lax.bitcast_convert_type(res3[:, 1, :].astype(_BF16), jnp.uint16)
    word = lo.astype(jnp.uint32) | (hi.astype(jnp.uint32) << 16)
    o_ref[...] = lax.bitcast_convert_type(word, _I32)


def _rows_mm_bias_pack(a, w, b2d, block_rows):
    m, k = a.shape
    h = w.shape[1]
    assert m % block_rows == 0 and block_rows % 2 == 0
    return pl.pallas_call(
        _mm_pack_body,
        grid=(m // block_rows,),
        in_specs=[
            pl.BlockSpec((block_rows, k), lambda i: (i, 0)),
            pl.BlockSpec((k, h), lambda i: (0, 0)),
            pl.BlockSpec((1, h), lambda i: (0, 0)),
        ],
        out_specs=pl.BlockSpec((block_rows // 2, h), lambda i: (i, 0)),
        out_shape=jax.ShapeDtypeStruct((m // 2, h), _I32),
    )(a, w, b2d)


def _emb_body(x_ref, a0_ref, a1_ref, w1_ref, w2_ref, b1_ref,
              wps_ref, wpo_ref, g1_ref, g2_ref):
    aggr = a0_ref[0] + a1_ref[0]
    emb = jnp.maximum(
        jnp.dot(x_ref[...], w1_ref[...], preferred_element_type=_F32)
        + jnp.dot(aggr, w2_ref[...], preferred_element_type=_F32)
        + b1_ref[...],
        0.0,
    )
    g1_ref[...] = jnp.dot(emb, wps_ref[...], preferred_element_type=_F32)
    g2_ref[...] = jnp.dot(emb, wpo_ref[...], preferred_element_type=_F32)


def _emb_g1_g2(x, aggr2, w1, w2, b1_2d, wps, wpo, block_rows):
    n, d = x.shape
    h = wps.shape[1]
    assert n % block_rows == 0
    full = lambda i: (0, 0)
    row_blk = lambda i: (i, 0)
    return pl.pallas_call(
        _emb_body,
        grid=(n // block_rows,),
        in_specs=[
            pl.BlockSpec((block_rows, d), row_blk),
            pl.BlockSpec((1, block_rows, d), lambda i: (0, i, 0)),
            pl.BlockSpec((1, block_rows, d), lambda i: (1, i, 0)),
            pl.BlockSpec((d, d), full),
            pl.BlockSpec((d, d), full),
            pl.BlockSpec((1, d), full),
            pl.BlockSpec((d, h), full),
            pl.BlockSpec((d, h), full),
        ],
        out_specs=[
            pl.BlockSpec((block_rows, h), row_blk),
            pl.BlockSpec((block_rows, h), row_blk),
        ],
        out_shape=[
            jax.ShapeDtypeStruct((n, h), _F32),
            jax.ShapeDtypeStruct((n, h), _F32),
        ],
    )(x, aggr2, aggr2, w1, w2, b1_2d, wps, wpo)


def _make_stage_a(n_nodes, e, d):
    assert e % _CA == 0
    nch = e // _CA
    nka = (nch + _NW - 1) // _NW
    npos = nka + 2
    npos += (-npos) % 3
    kcol = d // _L
    nfull = n_nodes // _CA
    ntail = n_nodes % _CA

    @functools.partial(
        pl.kernel,
        out_type=jax.ShapeDtypeStruct((2 * n_nodes, d), _F32),
        mesh=plsc.VectorSubcoreMesh(core_axis_name="c", subcore_axis_name="s"),
        scratch_types=[
            pltpu.VMEM((3, _CA, d), _F32),
            pltpu.VMEM((3, _CA, d), _F32),
            pltpu.VMEM((3, _CA), _I32),
            pltpu.VMEM((3, _CA), _I32),
            pltpu.VMEM_SHARED((n_nodes, d), _F32),
        ] + [pltpu.SemaphoreType.DMA] * 12,
        compiler_params=pltpu.CompilerParams(needs_layout_passes=False),
    )
    def stage_a(x_hbm, src_hbm, dst_hbm, p_hbm, z_hbm, out_hbm,
                pbuf, xbuf, sidx, didx, aggr_s, *sems):
        semi = sems[0:3]
        semp = sems[3:6]
        semg = sems[6:9]
        semsc = sems[9:12]
        cid = lax.axis_index("c")
        sid = lax.axis_index("s")
        wid = sid * _NC + cid

        pltpu.sync_copy(z_hbm, xbuf.at[0])

        def zbody(kk, carry):
            chunk = sid + kk * _NS

            @pl.when(chunk < nfull)
            def _():
                pltpu.sync_copy(xbuf.at[0],
                                aggr_s.at[pl.ds(chunk * _CA, _CA), :])

            return carry

        lax.fori_loop(0, (nfull + _NS - 1) // _NS, zbody, 0)
        if ntail:
            @pl.when(sid == _NS - 1)
            def _():
                pltpu.sync_copy(xbuf.at[0, pl.ds(0, ntail), :],
                                aggr_s.at[pl.ds(nfull * _CA, ntail), :])
        plsc.subcore_barrier()

        def d_sidx(kk, u):
            base = (kk * _NW + wid) * _CA
            return pltpu.make_async_copy(
                src_hbm.at[pl.ds(base, _CA)], sidx.at[u], semi[u])

        def d_didx(kk, u):
            base = (kk * _NW + wid) * _CA
            return pltpu.make_async_copy(
                dst_hbm.at[pl.ds(base, _CA)], didx.at[u], semi[u])

        def d_p(kk, u):
            base = (kk * _NW + wid) * _CA
            return pltpu.make_async_copy(
                p_hbm.at[pl.ds(base, _CA), :], pbuf.at[u], semp[u])

        def d_gather(u):
            return pltpu.make_async_copy(
                x_hbm.at[sidx.at[u]], xbuf.at[u], semg[u])

        def d_scat(u):
            return pltpu.make_async_copy(
                xbuf.at[u], aggr_s.at[didx.at[u]], semsc[u])

        def s1(kk, u):
            c = kk * _NW + wid
            valid = c < nch

            @pl.when((kk >= 3) & valid)
            def _():
                d_scat(u).wait()

            @pl.when(valid)
            def _():
                d_sidx(kk, u).start()
                d_didx(kk, u).start()
                d_p(kk, u).start()

        def g(kk, u):
            c = kk * _NW + wid

            @pl.when((kk >= 0) & (c < nch))
            def _():
                d_sidx(kk, u).wait()
                d_didx(kk, u).wait()
                d_gather(u).start()

        def f(kk, u):
            c = kk * _NW + wid

            @pl.when((kk >= 0) & (c < nch))
            def _():
                d_gather(u).wait()
                d_p(kk, u).wait()

                def crow(row, carry2):
                    for k in range(kcol):
                        sl = pl.ds(k * _L, _L)
                        v = xbuf[u, row, sl] + pbuf[u, row, sl]
                        xbuf[u, row, sl] = jnp.maximum(v, 0.0)
                    return carry2

                lax.fori_loop(0, _CA, crow, 0)
                pltpu.async_copy(xbuf.at[u], aggr_s.at[didx.at[u]],
                                 semsc[u], add=True)

        def mainloop(q, carry):
            i0 = q * 3
            for uu in range(3):
                i = i0 + uu
                g(i - 1, (uu + 2) % 3)
                s1(i, uu)
                f(i - 2, (uu + 1) % 3)
            return carry

        lax.fori_loop(0, npos // 3, mainloop, 0)
        for u in range(3):
            d_scat(u).wait()
        plsc.subcore_barrier()

        def obody(kk, carry):
            chunk = sid + kk * _NS

            @pl.when(chunk < nfull)
            def _():
                pltpu.sync_copy(
                    aggr_s.at[pl.ds(chunk * _CA, _CA), :],
                    out_hbm.at[pl.ds(cid * n_nodes + chunk * _CA, _CA), :])

            return carry

        lax.fori_loop(0, (nfull + _NS - 1) // _NS, obody, 0)
        if ntail:
            @pl.when(sid == _NS - 1)
            def _():
                pltpu.sync_copy(
                    aggr_s.at[pl.ds(nfull * _CA, ntail), :],
                    out_hbm.at[pl.ds(cid * n_nodes + nfull * _CA, ntail), :])

    return stage_a


def _make_stage_b(t, d):
    nfullb = t // _CB
    tb = t % _CB
    assert tb % 8 == 0
    ntotb = nfullb + (1 if tb else 0)
    nkb = (ntotb + _NW - 1) // _NW
    npos = nkb + 2
    npos += (-npos) % 3
    kcol = d // _L
    tailw = nfullb % _NW
    tail_slot = ((ntotb - 1 - tailw) // _NW) % 3

    @functools.partial(
        pl.kernel,
        out_type=jax.ShapeDtypeStruct((t,), _F32),
        mesh=plsc.VectorSubcoreMesh(core_axis_name="c", subcore_axis_name="s"),
        scratch_types=[
            pltpu.VMEM((3, _CB, d), _F32),
            pltpu.VMEM((3, _CB, d), _F32),
            pltpu.VMEM((3, _CB, d), _F32),
            pltpu.VMEM((3, _CB), _I32),
            pltpu.VMEM((3, _CB), _I32),
            pltpu.VMEM((3, _CB), _F32),
            pltpu.VMEM((d + _L,), _F32),
        ] + [pltpu.SemaphoreType.DMA] * 15,
        compiler_params=pltpu.CompilerParams(needs_layout_passes=False),
    )
    def stage_b(g1_hbm, g2_hbm, s_hbm, o_hbm, r_hbm, wb_hbm, out_hbm,
                g1buf, g2buf, rbuf, sidx, oidx, obuf, wbuf, *sems):
        semi = sems[0:3]
        semr = sems[3:6]
        semg1 = sems[6:9]
        semg2 = sems[9:12]
        semo = sems[12:15]
        cid = lax.axis_index("c")
        sid = lax.axis_index("s")
        wid = sid * _NC + cid

        pltpu.sync_copy(wb_hbm, wbuf)
        w2 = [wbuf[pl.ds(k * _L, _L)] for k in range(kcol)]
        b2v = wbuf[pl.ds(d, _L)]
        iota = lax.iota(_I32, _L)
        zero = jnp.zeros((_L,), _F32)

        def d_sidx(kk, u, n):
            base = (kk * _NW + wid) * _CB
            return pltpu.make_async_copy(
                s_hbm.at[pl.ds(base, n)],
                sidx.at[u, pl.ds(0, n)], semi[u])

        def d_oidx(kk, u, n):
            base = (kk * _NW + wid) * _CB
            return pltpu.make_async_copy(
                o_hbm.at[pl.ds(base, n)],
                oidx.at[u, pl.ds(0, n)], semi[u])

        def d_r(kk, u, n):
            base = (kk * _NW + wid) * _CB
            return pltpu.make_async_copy(
                r_hbm.at[pl.ds(base, n), :],
                rbuf.at[u, pl.ds(0, n), :], semr[u])

        def d_g1(u, n):
            return pltpu.make_async_copy(
                g1_hbm.at[sidx.at[u, pl.ds(0, n)]],
                g1buf.at[u, pl.ds(0, n), :], semg1[u])

        def d_g2(u, n):
            return pltpu.make_async_copy(
                g2_hbm.at[oidx.at[u, pl.ds(0, n)]],
                g2buf.at[u, pl.ds(0, n), :], semg2[u])

        def d_o(kk, u, n):
            base = (kk * _NW + wid) * _CB
            return pltpu.make_async_copy(
                obuf.at[u, pl.ds(0, n)],
                out_hbm.at[pl.ds(base, n)], semo[u])

        def both(cond_full, cond_tail, fn):
            if tb:
                @pl.when(cond_full)
                def _():
                    fn(_CB)

                @pl.when(cond_tail)
                def _():
                    fn(tb)
            else:
                @pl.when(cond_full)
                def _():
                    fn(_CB)

        def s1(kk, u):
            c = kk * _NW + wid

            @pl.when((kk >= 3) & (c < ntotb))
            def _():
                d_o(kk - 3, u, _CB).wait()

            def start(n):
                d_sidx(kk, u, n).start()
                d_oidx(kk, u, n).start()
                d_r(kk, u, n).start()

            both(c < nfullb, c == nfullb, start)

        def g(kk, u):
            ok = kk >= 0
            c = kk * _NW + wid

            def fire(n):
                d_sidx(kk, u, n).wait()
                d_oidx(kk, u, n).wait()
                d_g1(u, n).start()
                d_g2(u, n).start()

            both(ok & (c < nfullb), ok & (c == nfullb), fire)

        def f(kk, u):
            ok = kk >= 0
            c = kk * _NW + wid

            def waits(n):
                d_g1(u, n).wait()
                d_g2(u, n).wait()
                d_r(kk, u, n).wait()

            both(ok & (c < nfullb), ok & (c == nfullb), waits)

            @pl.when(ok & (c < ntotb))
            def _():
                nrows = jnp.where(c == nfullb, tb, _CB) if tb else _CB

                def ebody(row, s):
                    acc = zero
                    for k in range(kcol):
                        sl = pl.ds(k * _L, _L)
                        v = (g1buf[u, row, sl] + g2buf[u, row, sl]
                             + rbuf[u, row, sl])
                        acc = acc + jnp.maximum(v, 0.0) * w2[k]
                    tot = jnp.sum(acc)
                    lane = lax.rem(row, _L)
                    s = jnp.where(iota == lane, tot, s)

                    @pl.when(lane == _L - 1)
                    def _():
                        obuf[u, pl.ds(row - (_L - 1), _L)] = s

                    return jnp.where(lane == _L - 1, b2v, s)

                lax.fori_loop(0, nrows, ebody, b2v)

            both(ok & (c < nfullb), ok & (c == nfullb),
                 lambda n: d_o(kk, u, n).start())

        def mainloop(q, carry):
            i0 = q * 3
            for uu in range(3):
                i = i0 + uu
                g(i - 1, (uu + 2) % 3)
                s1(i, uu)
                f(i - 2, (uu + 1) % 3)
            return carry

        lax.fori_loop(0, npos // 3, mainloop, 0)
        for u in range(3):
            if tb and u == tail_slot:
                @pl.when(wid == tailw)
                def _():
                    d_o(0, u, tb).wait()

                @pl.when(wid != tailw)
                def _():
                    d_o(0, u, _CB).wait()
            else:
                d_o(0, u, _CB).wait()

    return stage_b


def kernel(x, edge_index, edge_attr, target_edges, target_edge_attrs,
           We, W1, W2, b1, Wp1, bp1, Wp2, bp2):
    n_nodes, d = x.shape
    h = Wp1.shape[1]
    e = edge_index.shape[1]
    t = target_edges.shape[1]

    ei = edge_index.astype(_I32)
    te = target_edges.astype(_I32)

    wp1_s = Wp1[:d]
    wp1_m = Wp1[d:d + target_edge_attrs.shape[1]]
    wp1_o = Wp1[d + target_edge_attrs.shape[1]:]

    p = _rows_mm_bias(edge_attr, We, jnp.zeros((1, d), _F32),
                      block_rows=4000, out_dtype=_F32)
    r = _rows_mm_bias(target_edge_attrs, wp1_m, bp1.reshape(1, h),
                      block_rows=4000, out_dtype=_F32)

    stage_a = _make_stage_a(n_nodes, e, d)
    zeros_chunk = jnp.zeros((_CA, d), _F32)
    aggr2 = stage_a(x, ei[0], ei[1], p, zeros_chunk).reshape(2, n_nodes, d)

    g1, g2 = _emb_g1_g2(x, aggr2, W1, W2, b1.reshape(1, d),
                        wp1_s, wp1_o, block_rows=400)

    wb = jnp.concatenate([Wp2[:, 0], jnp.broadcast_to(bp2, (_L,))])
    stage_b = _make_stage_b(t, h)
    return stage_b(g1, g2, te[0], te[1], r, wb)


# speedup vs baseline: 1.3044x; 1.0165x over previous
import functools

import numpy as _np

import jax
import jax.numpy as jnp
from jax import lax
from jax.experimental import pallas as pl
from jax.experimental.pallas import tpu as pltpu
from jax.experimental.pallas import tpu_sc as plsc

_NC = 2
_NS = 16
_NW = _NC * _NS
_L = 16
_CA = 64
_CB = 96

_F32 = jnp.float32
_BF16 = jnp.bfloat16
_I32 = jnp.int32
_ILV = plsc.PackFormat.INTERLEAVED


def _mm_bias_body(a_ref, w_ref, b_ref, o_ref):
    o_ref[...] = (
        jnp.dot(a_ref[...], w_ref[...], preferred_element_type=_F32)
        + b_ref[...]
    ).astype(o_ref.dtype)


def _rows_mm_bias(a, w, b2d, block_rows, out_dtype):
    m, k = a.shape
    h = w.shape[1]
    assert m % block_rows == 0
    return pl.pallas_call(
        _mm_bias_body,
        grid=(m // block_rows,),
        in_specs=[
            pl.BlockSpec((block_rows, k), lambda i: (i, 0)),
            pl.BlockSpec((k, h), lambda i: (0, 0)),
            pl.BlockSpec((1, h), lambda i: (0, 0)),
        ],
        out_specs=pl.BlockSpec((block_rows, h), lambda i: (i, 0)),
        out_shape=jax.ShapeDtypeStruct((m, h), out_dtype),
    )(a, w, b2d)


def _mm_pack_body(a_ref, w_ref, b_ref, o_ref):
    res = (jnp.dot(a_ref[...], w_ref[...], preferred_element_type=_F32)
           + b_ref[...])
    res3 = res.reshape(res.shape[0] // 2, 2, res.shape[1])
    lo = lax.bitcast_convert_type(res3[:, 0, :].astype(_BF16), jnp.uint16)
    hi = lax.bitcast_convert_type(res3[:, 1, :].astype(_BF16), jnp.uint16)
    word = lo.astype(jnp.uint32) | (hi.astype(jnp.uint32) << 16)
    o_ref[...] = lax.bitcast_convert_type(word, _I32)


def _rows_mm_bias_pack(a, w, b2d, block_rows):
    m, k = a.shape
    h = w.shape[1]
    assert m % block_rows == 0 and block_rows % 2 == 0
    return pl.pallas_call(
        _mm_pack_body,
        grid=(m // block_rows,),
        in_specs=[
            pl.BlockSpec((block_rows, k), lambda i: (i, 0)),
            pl.BlockSpec((k, h), lambda i: (0, 0)),
            pl.BlockSpec((1, h), lambda i: (0, 0)),
        ],
        out_specs=pl.BlockSpec((block_rows // 2, h), lambda i: (i, 0)),
        out_shape=jax.ShapeDtypeStruct((m // 2, h), _I32),
    )(a, w, b2d)


def _emb_body(x_ref, a0_ref, a1_ref, w1_ref, w2_ref, b1_ref,
              wps_ref, wpo_ref, g1_ref, g2_ref):
    aggr = a0_ref[0] + a1_ref[0]
    emb = jnp.maximum(
        jnp.dot(x_ref[...], w1_ref[...], preferred_element_type=_F32)
        + jnp.dot(aggr, w2_ref[...], preferred_element_type=_F32)
        + b1_ref[...],
        0.0,
    )
    g1_ref[...] = jnp.dot(emb, wps_ref[...], preferred_element_type=_F32)
    g2_ref[...] = jnp.dot(emb, wpo_ref[...], preferred_element_type=_F32)


def _emb_g1_g2(x, aggr2, w1, w2, b1_2d, wps, wpo, block_rows):
    n, d = x.shape
    h = wps.shape[1]
    assert n % block_rows == 0
    full = lambda i: (0, 0)
    row_blk = lambda i: (i, 0)
    return pl.pallas_call(
        _emb_body,
        grid=(n // block_rows,),
        in_specs=[
            pl.BlockSpec((block_rows, d), row_blk),
            pl.BlockSpec((1, block_rows, d), lambda i: (0, i, 0)),
            pl.BlockSpec((1, block_rows, d), lambda i: (1, i, 0)),
            pl.BlockSpec((d, d), full),
            pl.BlockSpec((d, d), full),
            pl.BlockSpec((1, d), full),
            pl.BlockSpec((d, h), full),
            pl.BlockSpec((d, h), full),
        ],
        out_specs=[
            pl.BlockSpec((block_rows, h), row_blk),
            pl.BlockSpec((block_rows, h), row_blk),
        ],
        out_shape=[
            jax.ShapeDtypeStruct((n, h), _F32),
            jax.ShapeDtypeStruct((n, h), _F32),
        ],
    )(x, aggr2, aggr2, w1, w2, b1_2d, wps, wpo)


def _make_stage_a(n_nodes, e, d):
    assert e % _CA == 0
    nch = e // _CA
    nka = (nch + _NW - 1) // _NW
    npos = nka + 2
    npos += (-npos) % 3
    kcol = d // _L
    nfull = n_nodes // _CA
    ntail = n_nodes % _CA

    @functools.partial(
        pl.kernel,
        out_type=jax.ShapeDtypeStruct((2 * n_nodes, d), _F32),
        mesh=plsc.VectorSubcoreMesh(core_axis_name="c", subcore_axis_name="s"),
        scratch_types=[
            pltpu.VMEM((3, _CA, d), _F32),
            pltpu.VMEM((3, _CA, d), _F32),
            pltpu.VMEM((3, _CA), _I32),
            pltpu.VMEM((3, _CA), _I32),
            pltpu.VMEM_SHARED((n_nodes, d), _F32),
        ] + [pltpu.SemaphoreType.DMA] * 12,
        compiler_params=pltpu.CompilerParams(needs_layout_passes=False),
    )
    def stage_a(x_hbm, src_hbm, dst_hbm, p_hbm, z_hbm, out_hbm,
                pbuf, xbuf, sidx, didx, aggr_s, *sems):
        semi = sems[0:3]
        semp = sems[3:6]
        semg = sems[6:9]
        semsc = sems[9:12]
        cid = lax.axis_index("c")
        sid = lax.axis_index("s")
        wid = sid * _NC + cid

        pltpu.sync_copy(z_hbm, xbuf.at[0])

        def zbody(kk, carry):
            chunk = sid + kk * _NS

            @pl.when(chunk < nfull)
            def _():
                pltpu.sync_copy(xbuf.at[0],
                                aggr_s.at[pl.ds(chunk * _CA, _CA), :])

            return carry

        lax.fori_loop(0, (nfull + _NS - 1) // _NS, zbody, 0)
        if ntail:
            @pl.when(sid == _NS - 1)
            def _():
                pltpu.sync_copy(xbuf.at[0, pl.ds(0, ntail), :],
                                aggr_s.at[pl.ds(nfull * _CA, ntail), :])
        plsc.subcore_barrier()

        def d_sidx(kk, u):
            base = (kk * _NW + wid) * _CA
            return pltpu.make_async_copy(
                src_hbm.at[pl.ds(base, _CA)], sidx.at[u], semi[u])

        def d_didx(kk, u):
            base = (kk * _NW + wid) * _CA
            return pltpu.make_async_copy(
                dst_hbm.at[pl.ds(base, _CA)], didx.at[u], semi[u])

        def d_p(kk, u):
            base = (kk * _NW + wid) * _CA
            return pltpu.make_async_copy(
                p_hbm.at[pl.ds(base, _CA), :], pbuf.at[u], semp[u])

        def d_gather(u):
            return pltpu.make_async_copy(
                x_hbm.at[sidx.at[u]], xbuf.at[u], semg[u])

        def d_scat(u):
            return pltpu.make_async_copy(
                xbuf.at[u], aggr_s.at[didx.at[u]], semsc[u])

        def s1(kk, u):
            c = kk * _NW + wid
            valid = c < nch

            @pl.when((kk >= 3) & valid)
            def _():
                d_scat(u).wait()

            @pl.when(valid)
            def _():
                d_sidx(kk, u).start()
                d_didx(kk, u).start()
                d_p(kk, u).start()

        def g(kk, u):
            c = kk * _NW + wid

            @pl.when((kk >= 0) & (c < nch))
            def _():
                d_sidx(kk, u).wait()
                d_didx(kk, u).wait()
                d_gather(u).start()

        def f(kk, u):
            c = kk * _NW + wid

            @pl.when((kk >= 0) & (c < nch))
            def _():
                d_gather(u).wait()
                d_p(kk, u).wait()

                def crow(row, carry2):
                    for k in range(kcol):
                        sl = pl.ds(k * _L, _L)
                        v = xbuf[u, row, sl] + pbuf[u, row, sl]
                        xbuf[u, row, sl] = jnp.maximum(v, 0.0)
                    return carry2

                lax.fori_loop(0, _CA, crow, 0)
                pltpu.async_copy(xbuf.at[u], aggr_s.at[didx.at[u]],
                                 semsc[u], add=True)

        def mainloop(q, carry):
            i0 = q * 3
            for uu in range(3):
                i = i0 + uu
                g(i - 1, (uu + 2) % 3)
                s1(i, uu)
                f(i - 2, (uu + 1) % 3)
            return carry

        lax.fori_loop(0, npos // 3, mainloop, 0)
        for u in range(3):
            d_scat(u).wait()
        plsc.subcore_barrier()

        def obody(kk, carry):
            chunk = sid + kk * _NS

            @pl.when(chunk < nfull)
            def _():
                pltpu.sync_copy(
                    aggr_s.at[pl.ds(chunk * _CA, _CA), :],
                    out_hbm.at[pl.ds(cid * n_nodes + chunk * _CA, _CA), :])

            return carry

        lax.fori_loop(0, (nfull + _NS - 1) // _NS, obody, 0)
        if ntail:
            @pl.when(sid == _NS - 1)
            def _():
                pltpu.sync_copy(
                    aggr_s.at[pl.ds(nfull * _CA, ntail), :],
                    out_hbm.at[pl.ds(cid * n_nodes + nfull * _CA, ntail), :])

    return stage_a


def _make_stage_b(t, d):
    nfullb = t // _CB
    tb = t % _CB
    assert tb % 8 == 0
    ntotb = nfullb + (1 if tb else 0)
    nkb = (ntotb + _NW - 1) // _NW
    npos = nkb + 2
    npos += (-npos) % 3
    kcol = d // _L
    tailw = nfullb % _NW
    tail_slot = ((ntotb - 1 - tailw) // _NW) % 3

    @functools.partial(
        pl.kernel,
        out_type=jax.ShapeDtypeStruct((t,), _F32),
        mesh=plsc.VectorSubcoreMesh(core_axis_name="c", subcore_axis_name="s"),
        scratch_types=[
            pltpu.VMEM((3, _CB, d), _F32),
            pltpu.VMEM((3, _CB, d), _F32),
            pltpu.VMEM((3, _CB, d), _F32),
            pltpu.VMEM((3, _CB), _I32),
            pltpu.VMEM((3, _CB), _I32),
            pltpu.VMEM((3, _CB), _F32),
            pltpu.VMEM((d + _L,), _F32),
        ] + [pltpu.SemaphoreType.DMA] * 15,
        compiler_params=pltpu.CompilerParams(needs_layout_passes=False),
    )
    def stage_b(g1_hbm, g2_hbm, s_hbm, o_hbm, r_hbm, wb_hbm, out_hbm,
                g1buf, g2buf, rbuf, sidx, oidx, obuf, wbuf, *sems):
        semi = sems[0:3]
        semr = sems[3:6]
        semg1 = sems[6:9]
        semg2 = sems[9:12]
        semo = sems[12:15]
        cid = lax.axis_index("c")
        sid = lax.axis_index("s")
        wid = sid * _NC + cid

        pltpu.sync_copy(wb_hbm, wbuf)
        w2 = [wbuf[pl.ds(k * _L, _L)] for k in range(kcol)]
        b2v = wbuf[pl.ds(d, _L)]
        iota = lax.iota(_I32, _L)
        zero = jnp.zeros((_L,), _F32)

        def d_sidx(kk, u, n):
            base = (kk * _NW + wid) * _CB
            return pltpu.make_async_copy(
                s_hbm.at[pl.ds(base, n)],
                sidx.at[u, pl.ds(0, n)], semi[u])

        def d_oidx(kk, u, n):
            base = (kk * _NW + wid) * _CB
            return pltpu.make_async_copy(
                o_hbm.at[pl.ds(base, n)],
                oidx.at[u, pl.ds(0, n)], semi[u])

        def d_r(kk, u, n):
            base = (kk * _NW + wid) * _CB
            return pltpu.make_async_copy(
                r_hbm.at[pl.ds(base, n), :],
                rbuf.at[u, pl.ds(0, n), :], semr[u])

        def d_g1(u, n):
            return pltpu.make_async_copy(
                g1_hbm.at[sidx.at[u, pl.ds(0, n)]],
                g1buf.at[u, pl.ds(0, n), :], semg1[u])

        def d_g2(u, n):
            return pltpu.make_async_copy(
                g2_hbm.at[oidx.at[u, pl.ds(0, n)]],
                g2buf.at[u, pl.ds(0, n), :], semg2[u])

        def d_o(kk, u, n):
            base = (kk * _NW + wid) * _CB
            return pltpu.make_async_copy(
                obuf.at[u, pl.ds(0, n)],
                out_hbm.at[pl.ds(base, n)], semo[u])

        def both(cond_full, cond_tail, fn):
            if tb:
                @pl.when(cond_full)
                def _():
                    fn(_CB)

                @pl.when(cond_tail)
                def _():
                    fn(tb)
            else:
                @pl.when(cond_full)
                def _():
                    fn(_CB)

        def s1(kk, u):
            c = kk * _NW + wid

            @pl.when((kk >= 3) & (c < ntotb))
            def _():
                d_o(kk - 3, u, _CB).wait()

            def start(n):
                d_sidx(kk, u, n).start()
                d_oidx(kk, u, n).start()
                d_r(kk, u, n).start()

            both(c < nfullb, c == nfullb, start)

        def g(kk, u):
            ok = kk >= 0
            c = kk * _NW + wid

            def fire(n):
                d_sidx(kk, u, n).wait()
                d_oidx(kk, u, n).wait()
                d_g1(u, n).start()
                d_g2(u, n).start()

            both(ok & (c < nfullb), ok & (c == nfullb), fire)

        def f(kk, u):
            ok = kk >= 0
            c = kk * _NW + wid

            def waits(n):
                d_g1(u, n).wait()
                d_g2(u, n).wait()
                d_r(kk, u, n).wait()

            both(ok & (c < nfullb), ok & (c == nfullb), waits)

            @pl.when(ok & (c < ntotb))
            def _():
                nrows = jnp.where(c == nfullb, tb, _CB) if tb else _CB

                def ebody(row, s):
                    acc = zero
                    for k in range(kcol):
                        sl = pl.ds(k * _L, _L)
                        v = (g1buf[u, row, sl] + g2buf[u, row, sl]
                             + rbuf[u, row, sl])
                        acc = acc + jnp.maximum(v, 0.0) * w2[k]
                    tot = jnp.sum(acc)
                    lane = lax.rem(row, _L)
                    s = jnp.where(iota == lane, tot, s)

                    @pl.when(lane == _L - 1)
                    def _():
                        obuf[u, pl.ds(row - (_L - 1), _L)] = s

                    return jnp.where(lane == _L - 1, b2v, s)

                lax.fori_loop(0, nrows, ebody, b2v)

            both(ok & (c < nfullb), ok & (c == nfullb),
                 lambda n: d_o(kk, u, n).start())

        def mainloop(q, carry):
            i0 = q * 3
            for uu in range(3):
                i = i0 + uu
                g(i - 1, (uu + 2) % 3)
                s1(i, uu)
                f(i - 2, (uu + 1) % 3)
            return carry

        lax.fori_loop(0, npos // 3, mainloop, 0)
        for u in range(3):
            if tb and u == tail_slot:
                @pl.when(wid == tailw)
                def _():
                    d_o(0, u, tb).wait()

                @pl.when(wid != tailw)
                def _():
                    d_o(0, u, _CB).wait()
            else:
                d_o(0, u, _CB).wait()

    return stage_b


def kernel(x, edge_index, edge_attr, target_edges, target_edge_attrs,
           We, W1, W2, b1, Wp1, bp1, Wp2, bp2):
    n_nodes, d = x.shape
    h = Wp1.shape[1]
    e = edge_index.shape[1]
    t = target_edges.shape[1]

    ei = edge_index.astype(_I32)
    te = target_edges.astype(_I32)

    wp1_s = Wp1[:d]
    wp1_m = Wp1[d:d + target_edge_attrs.shape[1]]
    wp1_o = Wp1[d + target_edge_attrs.shape[1]:]

    p = _rows_mm_bias(edge_attr, We, jnp.zeros((1, d), _F32),
                      block_rows=8000, out_dtype=_F32)
    r = _rows_mm_bias(target_edge_attrs, wp1_m, bp1.reshape(1, h),
                      block_rows=8000, out_dtype=_F32)

    stage_a = _make_stage_a(n_nodes, e, d)
    zeros_chunk = jnp.zeros((_CA, d), _F32)
    aggr2 = stage_a(x, ei[0], ei[1], p, zeros_chunk).reshape(2, n_nodes, d)

    g1, g2 = _emb_g1_g2(x, aggr2, W1, W2, b1.reshape(1, d),
                        wp1_s, wp1_o, block_rows=400)

    wb = jnp.concatenate([Wp2[:, 0], jnp.broadcast_to(bp2, (_L,))])
    stage_b = _make_stage_b(t, h)
    return stage_b(g1, g2, te[0], te[1], r, wb)
